# baseline (speedup 1.0000x reference)
import functools

import jax
import jax.numpy as jnp
from jax import lax
from jax.experimental import pallas as pl
from jax.experimental.pallas import tpu as pltpu
from jax.experimental.pallas import tpu_sc as plsc

VOCAB = 1000
B_SZ = 1024
T_SZ = 50
NTOK = B_SZ * T_SZ
NW = 32
K_CH = 5
T_CH = T_SZ // K_CH
CTOK = B_SZ * T_CH
CPW = CTOK // NW
RC = 40
NRC = CPW // RC
TPW = NTOK // NW
SC_CH = 64
N_SCCH = TPW // SC_CH

_MESH = plsc.VectorSubcoreMesh(core_axis_name="c", subcore_axis_name="s")
_UNTILED = pltpu.CompilerParams(use_tc_tiling_on_sc=False)


def _row_lse_body(table_ref, out_ref):
    t = table_ref[...]
    m = jnp.max(t, axis=1, keepdims=True)
    s = jnp.sum(jnp.exp(t - m), axis=1, keepdims=True)
    out_ref[...] = jnp.log(s) + m


def _row_lse(table):
    out = pl.pallas_call(
        _row_lse_body,
        out_shape=jax.ShapeDtypeStruct((VOCAB, 1), jnp.float32),
    )(table)
    return out.reshape(VOCAB)


@functools.partial(
    pl.kernel,
    mesh=_MESH,
    compiler_params=_UNTILED,
    out_type=jax.ShapeDtypeStruct((CTOK, VOCAB), jnp.float32),
    scratch_types=[
        pltpu.VMEM((CPW,), jnp.int32),
        pltpu.VMEM((RC, VOCAB), jnp.float32),
        pltpu.VMEM((RC, VOCAB), jnp.float32),
        pltpu.SemaphoreType.DMA,
        pltpu.SemaphoreType.DMA,
    ],
)
def _sc_rows(x_hbm, table_hbm, out_hbm, xr, buf0, buf1, sg0, sg1):
    cid = lax.axis_index("c")
    sid = lax.axis_index("s")
    wid = sid * 2 + cid
    base = wid * CPW

    pltpu.sync_copy(x_hbm.at[pl.ds(base, CPW)], xr)

    def gather_start(j, buf, sem):
        idx = xr.at[pl.ds(j * RC, RC)]
        return pltpu.make_async_copy(table_hbm.at[idx], buf, sem)

    def write_out(buf, j):
        pltpu.sync_copy(buf, out_hbm.at[pl.ds(base + j * RC, RC)])

    gather_start(0, buf0, sg0).start()

    def body(g, carry):
        j = 2 * g
        gather_start(j, buf0, sg0).wait()
        gather_start(j + 1, buf1, sg1).start()
        write_out(buf0, j)
        gather_start(j + 1, buf1, sg1).wait()

        @pl.when(j + 2 < NRC)
        def _():
            gather_start(j + 2, buf0, sg0).start()

        write_out(buf1, j + 1)
        return carry

    lax.fori_loop(0, NRC // 2, body, 0)


@functools.partial(
    pl.kernel,
    mesh=_MESH,
    compiler_params=_UNTILED,
    out_type=jax.ShapeDtypeStruct((NW, 16), jnp.float32),
    scratch_types=[
        pltpu.VMEM((TPW,), jnp.int32),
        pltpu.VMEM((TPW,), jnp.int32),
        pltpu.VMEM((TPW,), jnp.int32),
        pltpu.VMEM((TPW,), jnp.float32),
        pltpu.VMEM((TPW,), jnp.float32),
        pltpu.VMEM((16,), jnp.float32),
        pltpu.SemaphoreType.DMA,
        pltpu.SemaphoreType.DMA,
    ],
)
def _sc_loss(x_hbm, y_hbm, tflat_hbm, lse_hbm, part_hbm,
             xs, yv, fv, lsev, pick, acc_v, sl, sp):
    cid = lax.axis_index("c")
    sid = lax.axis_index("s")
    wid = sid * 2 + cid
    base = wid * TPW

    pltpu.sync_copy(x_hbm.at[pl.ds(base, TPW)], xs)
    pltpu.sync_copy(y_hbm.at[pl.ds(base, TPW)], yv)

    def build_flat(i, carry):
        s16 = pl.ds(i * 16, 16)
        fv[s16] = xs[s16] * VOCAB + yv[s16]
        return carry

    lax.fori_loop(0, TPW // 16, build_flat, 0)

    def scalar_desc(i):
        s = pl.ds(i * SC_CH, SC_CH)
        dl = pltpu.make_async_copy(lse_hbm.at[xs.at[s]], lsev.at[s], sl)
        dp = pltpu.make_async_copy(tflat_hbm.at[fv.at[s]], pick.at[s], sp)
        return dl, dp

    def fire(i, carry):
        dl, dp = scalar_desc(i)
        dl.start()
        dp.start()
        return carry

    lax.fori_loop(0, N_SCCH, fire, 0)

    def drain(i, carry):
        dl, dp = scalar_desc(i)
        dl.wait()
        dp.wait()
        return carry

    lax.fori_loop(0, N_SCCH, drain, 0)

    def accum(i, a):
        s16 = pl.ds(i * 16, 16)
        return a + (lsev[s16] - pick[s16])

    acc = lax.fori_loop(0, TPW // 16, accum,
                        jnp.zeros((16,), jnp.float32))
    acc_v[...] = acc
    pltpu.sync_copy(acc_v, part_hbm.at[wid])


def _loss_body(part_ref, out_ref):
    out_ref[...] = jnp.sum(part_ref[...], keepdims=True) / NTOK


def _final_loss(partials):
    out = pl.pallas_call(
        _loss_body,
        out_shape=jax.ShapeDtypeStruct((1, 1), jnp.float32),
    )(partials)
    return out[0, 0]


def kernel(x, y, table):
    x32 = x.astype(jnp.int32)
    y32 = y.reshape(-1).astype(jnp.int32)
    table = table.astype(jnp.float32)
    tflat = jnp.concatenate(
        [table.reshape(-1), jnp.zeros((8,), jnp.float32)])
    row_lse = _row_lse(table)
    partials = _sc_loss(x32.reshape(-1), y32, tflat, row_lse)
    loss = _final_loss(partials)

    chunks = []
    for k in range(K_CH):
        xk = x32[:, k * T_CH:(k + 1) * T_CH].reshape(-1)
        chunks.append(_sc_rows(xk, table).reshape(B_SZ, T_CH, VOCAB))
    logits = jnp.concatenate(chunks, axis=1)
    return (logits, loss)


# speedup vs baseline: 3.2302x; 3.2302x over previous
import functools

import jax
import jax.numpy as jnp
from jax import lax
from jax.experimental import pallas as pl
from jax.experimental.pallas import tpu as pltpu
from jax.experimental.pallas import tpu_sc as plsc

VOCAB = 1000
B_SZ = 1024
T_SZ = 50
NTOK = B_SZ * T_SZ
NW = 32
TPW = NTOK // NW
RC = 50
NRC = TPW // RC
SC_CH = 64
N_SCCH = TPW // SC_CH

_MESH = plsc.VectorSubcoreMesh(core_axis_name="c", subcore_axis_name="s")
_UNTILED = pltpu.CompilerParams(use_tc_tiling_on_sc=False)


def _row_lse_body(table_ref, out_ref):
    t = table_ref[...]
    m = jnp.max(t, axis=1, keepdims=True)
    s = jnp.sum(jnp.exp(t - m), axis=1, keepdims=True)
    out_ref[...] = jnp.log(s) + m


def _row_lse(table):
    out = pl.pallas_call(
        _row_lse_body,
        out_shape=jax.ShapeDtypeStruct((VOCAB, 1), jnp.float32),
    )(table)
    return out.reshape(VOCAB)


@functools.partial(
    pl.kernel,
    mesh=_MESH,
    compiler_params=_UNTILED,
    out_type=jax.ShapeDtypeStruct((NTOK, VOCAB), jnp.float32),
    scratch_types=[
        pltpu.VMEM((NRC, RC), jnp.int32),
        pltpu.VMEM((RC, VOCAB), jnp.float32),
        pltpu.VMEM((RC, VOCAB), jnp.float32),
        pltpu.SemaphoreType.DMA,
        pltpu.SemaphoreType.DMA,
    ],
)
def _sc_rows(x2_hbm, table_hbm, out_hbm, xr, buf0, buf1, sg0, sg1):
    cid = lax.axis_index("c")
    sid = lax.axis_index("s")
    wid = sid * 2 + cid
    base = wid * TPW

    pltpu.sync_copy(x2_hbm.at[wid], xr)

    def gather_start(j, buf, sem):
        return pltpu.make_async_copy(table_hbm.at[xr.at[j]], buf, sem)

    def write_out(buf, j):
        pltpu.sync_copy(buf, out_hbm.at[pl.ds(base + j * RC, RC)])

    gather_start(0, buf0, sg0).start()

    def body(g, carry):
        j = 2 * g
        gather_start(j, buf0, sg0).wait()
        gather_start(j + 1, buf1, sg1).start()
        write_out(buf0, j)
        gather_start(j + 1, buf1, sg1).wait()

        @pl.when(j + 2 < NRC)
        def _():
            gather_start(j + 2, buf0, sg0).start()

        write_out(buf1, j + 1)
        return carry

    lax.fori_loop(0, NRC // 2, body, 0)


@functools.partial(
    pl.kernel,
    mesh=_MESH,
    compiler_params=_UNTILED,
    out_type=jax.ShapeDtypeStruct((NW, 16), jnp.float32),
    scratch_types=[
        pltpu.VMEM((TPW,), jnp.int32),
        pltpu.VMEM((TPW,), jnp.int32),
        pltpu.VMEM((TPW,), jnp.int32),
        pltpu.VMEM((TPW,), jnp.float32),
        pltpu.VMEM((TPW,), jnp.float32),
        pltpu.VMEM((16,), jnp.float32),
        pltpu.SemaphoreType.DMA,
        pltpu.SemaphoreType.DMA,
    ],
)
def _sc_loss(x_hbm, y_hbm, tflat_hbm, lse_hbm, part_hbm,
             xs, yv, fv, lsev, pick, acc_v, sl, sp):
    cid = lax.axis_index("c")
    sid = lax.axis_index("s")
    wid = sid * 2 + cid
    base = wid * TPW

    pltpu.sync_copy(x_hbm.at[pl.ds(base, TPW)], xs)
    pltpu.sync_copy(y_hbm.at[pl.ds(base, TPW)], yv)

    def build_flat(i, carry):
        s16 = pl.ds(i * 16, 16)
        fv[s16] = xs[s16] * VOCAB + yv[s16]
        return carry

    lax.fori_loop(0, TPW // 16, build_flat, 0)

    def scalar_desc(i):
        s = pl.ds(i * SC_CH, SC_CH)
        dl = pltpu.make_async_copy(lse_hbm.at[xs.at[s]], lsev.at[s], sl)
        dp = pltpu.make_async_copy(tflat_hbm.at[fv.at[s]], pick.at[s], sp)
        return dl, dp

    def fire(i, carry):
        dl, dp = scalar_desc(i)
        dl.start()
        dp.start()
        return carry

    lax.fori_loop(0, N_SCCH, fire, 0)

    def drain(i, carry):
        dl, dp = scalar_desc(i)
        dl.wait()
        dp.wait()
        return carry

    lax.fori_loop(0, N_SCCH, drain, 0)

    def accum(i, a):
        s16 = pl.ds(i * 16, 16)
        return a + (lsev[s16] - pick[s16])

    acc = lax.fori_loop(0, TPW // 16, accum,
                        jnp.zeros((16,), jnp.float32))
    acc_v[...] = acc
    pltpu.sync_copy(acc_v, part_hbm.at[wid])


def _loss_body(part_ref, out_ref):
    out_ref[...] = jnp.sum(part_ref[...], keepdims=True) / NTOK


def _final_loss(partials):
    out = pl.pallas_call(
        _loss_body,
        out_shape=jax.ShapeDtypeStruct((1, 1), jnp.float32),
    )(partials)
    return out[0, 0]


def kernel(x, y, table):
    B, T = x.shape
    x32 = x.astype(jnp.int32)
    y32 = y.reshape(-1).astype(jnp.int32)
    table = table.astype(jnp.float32)
    tflat = jnp.concatenate(
        [table.reshape(-1), jnp.zeros((8,), jnp.float32)])
    row_lse = _row_lse(table)
    logits_flat = _sc_rows(x32.reshape(NW, NRC, RC), table)
    partials = _sc_loss(x32.reshape(-1), y32, tflat, row_lse)
    loss = _final_loss(partials)
    return (logits_flat.reshape(B, T, VOCAB), loss)
